# baseline (speedup 1.0000x reference)
import functools

import jax
import jax.numpy as jnp
from jax import lax
from jax.experimental import pallas as pl
from jax.experimental.pallas import tpu as pltpu
from jax.experimental.pallas import tpu_sc as plsc

NC = 2
NS = 16
NW = NC * NS
L = 16

N = 100000
D = 64
C = 125
CP = 128
NCHUNK = 25
BPW = C * NCHUNK
NB = 4

_mesh = plsc.VectorSubcoreMesh(
    core_axis_name="c", subcore_axis_name="s", num_cores=NC, num_subcores=NS
)


@functools.partial(
    pl.kernel,
    out_type=jax.ShapeDtypeStruct((N, D), jnp.float32),
    mesh=_mesh,
    compiler_params=pltpu.CompilerParams(
        use_tc_tiling_on_sc=False, needs_layout_passes=False
    ),
    scratch_types=[
        pltpu.VMEM((BPW + 2 * L,), jnp.int32),
        pltpu.VMEM((NCHUNK, CP), jnp.int32),
        [pltpu.VMEM((CP, D), jnp.float32) for _ in range(NB)],
        [pltpu.SemaphoreType.DMA for _ in range(NB)],
        [pltpu.SemaphoreType.DMA for _ in range(NB)],
    ],
)
def _gather_kernel(idx_hbm, table_hbm, out_hbm, idx_s, idx_m, rows,
                   gsems, ssems):
    wid = lax.axis_index("s") * NC + lax.axis_index("c")
    base = wid * BPW
    abase = (base // 8) * 8
    shift = base - abase
    pltpu.sync_copy(idx_hbm.at[pl.ds(abase, BPW + L)], idx_s.at[pl.ds(0, BPW + L)])

    lanes = lax.iota(jnp.int32, L)

    @pl.loop(0, NCHUNK)
    def _repack(j):
        row = idx_m.at[j]
        for c in range(CP // L):
            v = plsc.load_gather(idx_s, [shift + j * C + c * L + lanes])
            if (c + 1) * L > C:
                v = jnp.where(c * L + lanes < C, v, 0)
            row[pl.ds(c * L, L)] = v

    def gather(j, b):
        return pltpu.make_async_copy(
            table_hbm.at[idx_m.at[j]], rows[b], gsems[b]
        )

    def store(j, b):
        return pltpu.make_async_copy(
            rows[b].at[pl.ds(0, C)],
            out_hbm.at[pl.ds(base + j * C, C)],
            ssems[b],
        )

    for b in range(NB):
        gather(b, b).start()

    for j in range(NCHUNK):
        b = j % NB
        gather(j, b).wait()
        store(j, b).start()
        if j + NB < NCHUNK:
            store(j, b).wait()
            gather(j + NB, b).start()

    for j in range(NCHUNK - NB, NCHUNK):
        store(j, j % NB).wait()


def kernel(x, table):
    table_b = lax.optimization_barrier(table)
    idx = jnp.pad(x.reshape(-1), (0, 2 * L))
    return _gather_kernel(idx, table_b)


# speedup vs baseline: 1.1036x; 1.1036x over previous
import functools

import jax
import jax.numpy as jnp
from jax import lax
from jax.experimental import pallas as pl
from jax.experimental.pallas import tpu as pltpu
from jax.experimental.pallas import tpu_sc as plsc

NC = 2
NS = 16
NW = NC * NS

N = 100000
D = 64
C = 125
NCHUNK = 25
BPW = C * NCHUNK
NB = 5

_mesh = plsc.VectorSubcoreMesh(
    core_axis_name="c", subcore_axis_name="s", num_cores=NC, num_subcores=NS
)


@functools.partial(
    pl.kernel,
    out_type=jax.ShapeDtypeStruct((N, D), jnp.float32),
    mesh=_mesh,
    compiler_params=pltpu.CompilerParams(use_tc_tiling_on_sc=False),
    scratch_types=[
        pltpu.VMEM((NCHUNK, C), jnp.int32),
        [pltpu.VMEM((C, D), jnp.float32) for _ in range(NB)],
        [pltpu.SemaphoreType.DMA for _ in range(NB)],
        [pltpu.SemaphoreType.DMA for _ in range(NB)],
    ],
)
def _gather_kernel(idx_hbm, table_hbm, out_hbm, idx_v, rows, gsems, ssems):
    wid = lax.axis_index("s") * NC + lax.axis_index("c")
    base = wid * BPW
    pltpu.sync_copy(idx_hbm.at[wid], idx_v)

    def gather(j, b):
        return pltpu.make_async_copy(table_hbm.at[idx_v.at[j]], rows[b], gsems[b])

    def store(j, b):
        return pltpu.make_async_copy(
            rows[b], out_hbm.at[pl.ds(base + j * C, C)], ssems[b]
        )

    for b in range(NB):
        gather(b, b).start()

    for j in range(NCHUNK):
        b = j % NB
        gather(j, b).wait()
        store(j, b).start()
        if j + NB < NCHUNK:
            store(j, b).wait()
            gather(j + NB, b).start()

    for j in range(NCHUNK - NB, NCHUNK):
        store(j, j % NB).wait()


def kernel(x, table):
    idx = x.reshape(NW, NCHUNK, C)
    return _gather_kernel(idx, table)
